# Initial kernel scaffold; baseline (speedup 1.0000x reference)
import jax
import jax.numpy as jnp
from jax.experimental import pallas as pl


def kernel(inputs, pos_embedding_table):
    raise NotImplementedError("write your pallas kernel here")


# speedup vs baseline: 1.6966x; 1.6966x over previous
import jax
import jax.numpy as jnp
from jax.experimental import pallas as pl


def _add_body(x_ref, t_ref, o_ref):
    o_ref[...] = x_ref[...] + t_ref[...][None]


def kernel(inputs, pos_embedding_table):
    B, S, D = inputs.shape
    BS = 512
    return pl.pallas_call(
        _add_body,
        grid=(S // BS, B),
        in_specs=[
            pl.BlockSpec((1, BS, D), lambda i, b: (b, i, 0)),
            pl.BlockSpec((BS, D), lambda i, b: (i, 0)),
        ],
        out_specs=pl.BlockSpec((1, BS, D), lambda i, b: (b, i, 0)),
        out_shape=jax.ShapeDtypeStruct((B, S, D), inputs.dtype),
    )(inputs, pos_embedding_table)
